# baseline (speedup 1.0000x reference)
import jax
import jax.numpy as jnp
from jax import lax
from jax.experimental import pallas as pl
from jax.experimental.pallas import tpu as pltpu
from jax.experimental.pallas import tpu_sc as plsc

N = 1024
NE = 3
C = 2
E = 32768
P = 4096
DH = 128
DS = C * DH
RB = 256

NSUB = 16
ECH = E // NSUB
ZCH = 65664
ACC = ZCH * NSUB
MCH = N * N // NSUB


def _sc_scatter_body(r_hbm, c_hbm, v_hbm, zeros_hbm, a_out, cs_out,
                     acc, rbuf, cbuf, vbuf, linbuf, csbuf):
    cid = lax.axis_index("c")
    sid = lax.axis_index("s")
    ebase = cid * (NE * E)
    abase = cid * (NE * N * N)
    cbase = cid * (NE * N)
    for j in range(NE):
        pltpu.sync_copy(zeros_hbm, acc.at[pl.ds(sid * ZCH, ZCH)])
        base = ebase + j * E + sid * ECH
        pltpu.sync_copy(r_hbm.at[pl.ds(base, ECH)], rbuf)
        pltpu.sync_copy(c_hbm.at[pl.ds(base, ECH)], cbuf)
        pltpu.sync_copy(v_hbm.at[pl.ds(base, ECH)], vbuf)

        def body(i, _):
            o = i * 16
            r = rbuf[pl.ds(o, 16)]
            c = cbuf[pl.ds(o, 16)]
            linbuf[pl.ds(o, 16)] = (r << 10) + c
            csbuf[pl.ds(o, 16)] = c + N * N
            return 0

        lax.fori_loop(0, ECH // 16, body, 0)
        plsc.subcore_barrier()
        pltpu.sync_copy(vbuf, acc.at[linbuf], add=True)
        pltpu.sync_copy(vbuf, acc.at[csbuf], add=True)
        plsc.subcore_barrier()
        pltpu.sync_copy(acc.at[pl.ds(sid * MCH, MCH)],
                        a_out.at[pl.ds(abase + j * N * N + sid * MCH, MCH)])

        @pl.when(sid == 0)
        def _():
            pltpu.sync_copy(acc.at[pl.ds(N * N, N)],
                            cs_out.at[pl.ds(cbase + j * N, N)])

        plsc.subcore_barrier()


@jax.jit
def _sc_scatter(rows, cols, vals):
    zeros = jnp.zeros((ZCH,), jnp.float32)
    mesh = plsc.VectorSubcoreMesh(core_axis_name="c", subcore_axis_name="s")
    f = pl.kernel(
        _sc_scatter_body,
        mesh=mesh,
        out_type=[
            jax.ShapeDtypeStruct((2 * NE * N * N,), jnp.float32),
            jax.ShapeDtypeStruct((2 * NE * N,), jnp.float32),
        ],
        scratch_types=[
            pltpu.VMEM_SHARED((ACC,), jnp.float32),
            pltpu.VMEM((ECH,), jnp.int32),
            pltpu.VMEM((ECH,), jnp.int32),
            pltpu.VMEM((ECH,), jnp.float32),
            pltpu.VMEM((ECH,), jnp.int32),
            pltpu.VMEM((ECH,), jnp.int32),
        ],
    )
    return f(rows, cols, vals, zeros)


GCH = (2 * P) // (2 * NSUB)


def _sc_gather_body(tab, idx, out, ibuf, rows, sem):
    cid = lax.axis_index("c")
    sid = lax.axis_index("s")
    base = (cid * NSUB + sid) * GCH
    pltpu.sync_copy(idx.at[pl.ds(base, GCH)], ibuf)
    pltpu.async_copy(tab.at[ibuf], rows, sem).wait()
    pltpu.sync_copy(rows, out.at[pl.ds(base, GCH)])


@jax.jit
def _sc_gather(table, catidx):
    mesh = plsc.VectorSubcoreMesh(core_axis_name="c", subcore_axis_name="s")
    f = pl.kernel(
        _sc_gather_body,
        mesh=mesh,
        out_type=jax.ShapeDtypeStruct((2 * P, DS), jnp.float32),
        scratch_types=[
            pltpu.VMEM((GCH,), jnp.int32),
            pltpu.VMEM((GCH, DS), jnp.float32),
            pltpu.SemaphoreType.DMA,
        ],
    )
    return f(table, catidx)


NIO = N // RB


def _main_body(a_ref, cs_ref, f1_ref, f2c_ref, s1_ref, s2_ref, xt_ref,
               wt_ref, b_ref, out_ref, racc, vblk, dinv, zt, t1t, sacc):
    p = pl.program_id(1)
    io = pl.program_id(2)
    j = pl.program_id(3)

    @pl.when((p == 0) & (io == 0) & (j == 0))
    def _():
        racc[...] = jnp.dot(f1_ref[0], cs_ref[0],
                            preferred_element_type=jnp.float32)

    ablk = a_ref[0, 0]

    @pl.when(p == 0)
    def _():
        @pl.when(j == 0)
        def _():
            vblk[...] = jnp.zeros_like(vblk)

        res = jnp.dot(racc[...], ablk, preferred_element_type=jnp.float32)
        vblk[...] += res * f2c_ref[0, 0]

        @pl.when(j == NE - 1)
        def _():
            deg = 1.0 + vblk[...]
            dinvb = jnp.where(deg > 0.0, lax.rsqrt(deg), 0.0)
            dinv[io] = dinvb
            yt = jnp.dot(wt_ref[0], xt_ref[0],
                         preferred_element_type=jnp.float32)
            zt[io] = jnp.concatenate(
                [dinvb[0:1, :] * yt, dinvb[1:2, :] * yt], axis=0)

    def contract(scr):
        acc = None
        for kb in range(NIO):
            part = jnp.dot(scr[kb], ablk[kb * RB:(kb + 1) * RB, :],
                           preferred_element_type=jnp.float32)
            acc = part if acc is None else acc + part
        return acc

    @pl.when(p == 1)
    def _():
        @pl.when(j == 0)
        def _():
            sacc[...] = jnp.zeros_like(sacc)

        sacc[...] += contract(zt) * s1_ref[0, 0]

        @pl.when(j == NE - 1)
        def _():
            t1t[io] = sacc[...]

    @pl.when(p == 2)
    def _():
        @pl.when(j == 0)
        def _():
            sacc[...] = jnp.zeros_like(sacc)

        sacc[...] += contract(t1t) * s2_ref[0, 0]

        @pl.when(j == NE - 1)
        def _():
            dinvb = dinv[io]
            dcols = jnp.concatenate(
                [jnp.broadcast_to(dinvb[0:1, :], (DH, RB)),
                 jnp.broadcast_to(dinvb[1:2, :], (DH, RB))], axis=0)
            res = jnp.maximum(
                dcols * (sacc[...] + zt[io]) + b_ref[0], 0.0)
            out_ref[0] = res.T


def _main(Aall, csall, f1all, f2call, s1all, s2all, Xtall, Wtall, ball):
    grid = (2, 3, NIO, NE)
    return pl.pallas_call(
        _main_body,
        grid=grid,
        in_specs=[
            pl.BlockSpec((1, 1, N, RB), lambda s, p, io, j: (s, j, 0, io)),
            pl.BlockSpec((1, NE, N), lambda s, p, io, j: (s, 0, 0)),
            pl.BlockSpec((1, C, NE), lambda s, p, io, j: (s, 0, 0)),
            pl.BlockSpec((1, 1, C, 1), lambda s, p, io, j: (s, j, 0, 0)),
            pl.BlockSpec((1, 1, DS, 1), lambda s, p, io, j: (s, j, 0, 0)),
            pl.BlockSpec((1, 1, DS, 1), lambda s, p, io, j: (s, j, 0, 0)),
            pl.BlockSpec((1, DS, RB), lambda s, p, io, j: (s, 0, io)),
            pl.BlockSpec((1, DH, DS), lambda s, p, io, j: (s, 0, 0)),
            pl.BlockSpec((1, DS, 1), lambda s, p, io, j: (s, 0, 0)),
        ],
        out_specs=pl.BlockSpec((1, RB, DS), lambda s, p, io, j: (s, io, 0)),
        out_shape=jax.ShapeDtypeStruct((2, N, DS), jnp.float32),
        scratch_shapes=[
            pltpu.VMEM((C, N), jnp.float32),
            pltpu.VMEM((C, RB), jnp.float32),
            pltpu.VMEM((NIO, C, RB), jnp.float32),
            pltpu.VMEM((NIO, DS, RB), jnp.float32),
            pltpu.VMEM((NIO, DS, RB), jnp.float32),
            pltpu.VMEM((DS, RB), jnp.float32),
        ],
    )(Aall, csall, f1all, f2call, s1all, s2all, Xtall, Wtall, ball)


MB = 512


def _mlp_body(bu_ref, bv_ref, t_ref, m1a_ref, m1b_ref, b1_ref, m2_ref,
              b2_ref, m3_ref, b3_ref, bp_ref, loss_ref, lacc):
    i = pl.program_id(0)

    @pl.when(i == 0)
    def _():
        lacc[...] = jnp.zeros_like(lacc)

    h = jnp.dot(bu_ref[...], m1a_ref[...], preferred_element_type=jnp.float32)
    h += jnp.dot(bv_ref[...], m1b_ref[...], preferred_element_type=jnp.float32)
    h = jnp.maximum(h + b1_ref[...], 0.0)
    h = jnp.maximum(jnp.dot(h, m2_ref[...], preferred_element_type=jnp.float32)
                    + b2_ref[...], 0.0)
    logits = jnp.dot(h, m3_ref[...], preferred_element_type=jnp.float32) \
        + b3_ref[...]
    m = jnp.max(logits, axis=-1, keepdims=True)
    e = jnp.exp(logits - m)
    bp = e / jnp.sum(e, axis=-1, keepdims=True)
    bp_ref[...] = bp

    mm = jnp.max(bp, axis=-1, keepdims=True)
    lse = mm + jnp.log(jnp.sum(jnp.exp(bp - mm), axis=-1, keepdims=True))
    t = t_ref[...]
    bpt = bp[:, 0:1] * (1.0 - t) + bp[:, 1:2] * t
    lacc[...] += jnp.sum(lse - bpt, axis=0, keepdims=True)

    @pl.when(i == pl.num_programs(0) - 1)
    def _():
        loss_ref[...] = lacc[...] * (1.0 / P)


def _mlp(Bu, Bv, targetf, M1a, M1b, b1, M2, b2, M3, b3):
    grid = (P // MB,)
    return pl.pallas_call(
        _mlp_body,
        grid=grid,
        in_specs=[
            pl.BlockSpec((MB, DS), lambda i: (i, 0)),
            pl.BlockSpec((MB, DS), lambda i: (i, 0)),
            pl.BlockSpec((MB, 1), lambda i: (i, 0)),
            pl.BlockSpec((DS, DS), lambda i: (0, 0)),
            pl.BlockSpec((DS, DS), lambda i: (0, 0)),
            pl.BlockSpec((1, DS), lambda i: (0, 0)),
            pl.BlockSpec((DS, DS // 2), lambda i: (0, 0)),
            pl.BlockSpec((1, DS // 2), lambda i: (0, 0)),
            pl.BlockSpec((DS // 2, 2), lambda i: (0, 0)),
            pl.BlockSpec((1, 2), lambda i: (0, 0)),
        ],
        out_specs=[
            pl.BlockSpec((MB, 2), lambda i: (i, 0)),
            pl.BlockSpec((1, 1), lambda i: (0, 0)),
        ],
        out_shape=[
            jax.ShapeDtypeStruct((P, 2), jnp.float32),
            jax.ShapeDtypeStruct((1, 1), jnp.float32),
        ],
        scratch_shapes=[pltpu.VMEM((1, 1), jnp.float32)],
    )(Bu, Bv, targetf, M1a, M1b, b1, M2, b2, M3, b3)


def kernel(edge_index_u, edge_value_u, X_u, edge_index_v, edge_value_v, X_v,
           index_list, Wgt1_u, Wgt2_u, Wgt1_v, Wgt2_v, Wg_u, bg_u, Wg_v, bg_v,
           M1, b1, M2, b2, M3, b3):
    rows = jnp.concatenate([edge_index_u[:, 0, :].reshape(-1),
                            edge_index_v[:, 0, :].reshape(-1)]) \
        .astype(jnp.int32)
    cols = jnp.concatenate([edge_index_u[:, 1, :].reshape(-1),
                            edge_index_v[:, 1, :].reshape(-1)]) \
        .astype(jnp.int32)
    vals = jnp.concatenate([edge_value_u.reshape(-1),
                            edge_value_v.reshape(-1)])
    Afall, csfall = _sc_scatter(rows, cols, vals)
    Aall = Afall.reshape(2, NE, N, N)
    csall = csfall.reshape(2, NE, N)

    f1u = jax.nn.softmax(Wgt1_u, axis=1)
    f2u = jax.nn.softmax(Wgt2_u, axis=1)
    f1v = jax.nn.softmax(Wgt1_v, axis=1)
    f2v = jax.nn.softmax(Wgt2_v, axis=1)
    f1all = jnp.stack([f1u, f1v])
    f2call = jnp.stack([f2u.T, f2v.T])[..., None]
    s1all = jnp.repeat(jnp.stack([f1u.T, f1v.T]), DH, axis=2)[..., None]
    s2all = jnp.repeat(jnp.stack([f2u.T, f2v.T]), DH, axis=2)[..., None]
    Xtall = jnp.stack([X_u.T, X_v.T])
    Wtall = jnp.stack([Wg_u.T, Wg_v.T])
    ball = jnp.stack([jnp.tile(bg_u, (2,))[:, None],
                      jnp.tile(bg_v, (2,))[:, None]])

    Xout = _main(Aall, csall, f1all, f2call, s1all, s2all, Xtall, Wtall,
                 ball)
    Xu_ = Xout[0]
    Xv_ = Xout[1]

    u_idx = index_list[:, 0].astype(jnp.int32)
    v_idx = index_list[:, 1].astype(jnp.int32)
    target = index_list[:, 2]
    targetf = target.astype(jnp.float32)

    table = Xout.reshape(2 * N, DS)
    catidx = jnp.concatenate([u_idx, v_idx + N])
    Bcat = _sc_gather(table, catidx)
    Bu = Bcat[:P]
    Bv = Bcat[P:]

    Bp, loss2 = _mlp(Bu, Bv, targetf[:, None], M1[:DS], M1[DS:], b1[None, :],
                     M2, b2[None, :], M3, b3[None, :])
    loss = loss2.reshape(())
    return (Xu_, Xv_, f1u, f2u, f1v, f2v, loss, Bp, targetf)


# speedup vs baseline: 1.1561x; 1.1561x over previous
import jax
import jax.numpy as jnp
from jax import lax
from jax.experimental import pallas as pl
from jax.experimental.pallas import tpu as pltpu
from jax.experimental.pallas import tpu_sc as plsc

N = 1024
NE = 3
C = 2
E = 32768
P = 4096
DH = 128
DS = C * DH
RB = 256

NSUB = 16
ECH = E // NSUB
ZCH = 65664
ACC = ZCH * NSUB
MCH = N * N // NSUB


def _sc_scatter_body(r_hbm, c_hbm, v_hbm, zeros_hbm, a_out, cs_out,
                     acc, rbuf, cbuf, vbuf, linbuf, csbuf):
    cid = lax.axis_index("c")
    sid = lax.axis_index("s")
    ebase = cid * (NE * E)
    abase = cid * (NE * N * N)
    cbase = cid * (NE * N)
    for j in range(NE):
        pltpu.sync_copy(zeros_hbm, acc.at[pl.ds(sid * ZCH, ZCH)])
        base = ebase + j * E + sid * ECH
        pltpu.sync_copy(r_hbm.at[pl.ds(base, ECH)], rbuf)
        pltpu.sync_copy(c_hbm.at[pl.ds(base, ECH)], cbuf)
        pltpu.sync_copy(v_hbm.at[pl.ds(base, ECH)], vbuf)

        def body(i, _):
            o = i * 16
            r = rbuf[pl.ds(o, 16)]
            c = cbuf[pl.ds(o, 16)]
            linbuf[pl.ds(o, 16)] = ((c >> 7) << 17) + (r << 7) + (c & 127)
            csbuf[pl.ds(o, 16)] = c + N * N
            return 0

        lax.fori_loop(0, ECH // 16, body, 0)
        plsc.subcore_barrier()
        pltpu.sync_copy(vbuf, acc.at[linbuf], add=True)
        pltpu.sync_copy(vbuf, acc.at[csbuf], add=True)
        plsc.subcore_barrier()
        pltpu.sync_copy(acc.at[pl.ds(sid * MCH, MCH)],
                        a_out.at[pl.ds(abase + j * N * N + sid * MCH, MCH)])

        @pl.when(sid == 0)
        def _():
            pltpu.sync_copy(acc.at[pl.ds(N * N, N)],
                            cs_out.at[pl.ds(cbase + j * N, N)])

        plsc.subcore_barrier()


@jax.jit
def _sc_scatter(rows, cols, vals):
    zeros = jnp.zeros((ZCH,), jnp.float32)
    mesh = plsc.VectorSubcoreMesh(core_axis_name="c", subcore_axis_name="s")
    f = pl.kernel(
        _sc_scatter_body,
        mesh=mesh,
        out_type=[
            jax.ShapeDtypeStruct((2 * NE * N * N,), jnp.float32),
            jax.ShapeDtypeStruct((2 * NE * N,), jnp.float32),
        ],
        scratch_types=[
            pltpu.VMEM_SHARED((ACC,), jnp.float32),
            pltpu.VMEM((ECH,), jnp.int32),
            pltpu.VMEM((ECH,), jnp.int32),
            pltpu.VMEM((ECH,), jnp.float32),
            pltpu.VMEM((ECH,), jnp.int32),
            pltpu.VMEM((ECH,), jnp.int32),
        ],
    )
    return f(rows, cols, vals, zeros)


GCH = (2 * P) // (2 * NSUB)


def _sc_gather_body(tab, idx, out, ibuf, rows, sem):
    cid = lax.axis_index("c")
    sid = lax.axis_index("s")
    base = (cid * NSUB + sid) * GCH
    pltpu.sync_copy(idx.at[pl.ds(base, GCH)], ibuf)
    pltpu.async_copy(tab.at[ibuf], rows, sem).wait()
    pltpu.sync_copy(rows, out.at[pl.ds(base, GCH)])


@jax.jit
def _sc_gather(table, catidx):
    mesh = plsc.VectorSubcoreMesh(core_axis_name="c", subcore_axis_name="s")
    f = pl.kernel(
        _sc_gather_body,
        mesh=mesh,
        out_type=jax.ShapeDtypeStruct((2 * P, DS), jnp.float32),
        scratch_types=[
            pltpu.VMEM((GCH,), jnp.int32),
            pltpu.VMEM((GCH, DS), jnp.float32),
            pltpu.SemaphoreType.DMA,
        ],
    )
    return f(table, catidx)


NIO = N // RB
NG = RB // DH


def _main_body(a_ref, cs_ref, f1_ref, f2c_ref, s1_ref, s2_ref, xt_ref,
               wt_ref, b_ref, out_ref, racc, vblk, dinv, zt, t1t, sacc):
    p = pl.program_id(1)
    io = pl.program_id(2)
    j = pl.program_id(3)

    @pl.when((p == 0) & (io == 0) & (j == 0))
    def _():
        racc[...] = jnp.dot(f1_ref[0], cs_ref[0],
                            preferred_element_type=jnp.float32)

    @pl.when(p == 0)
    def _():
        @pl.when(j == 0)
        def _():
            vblk[...] = jnp.zeros_like(vblk)

        res = jnp.concatenate(
            [jnp.dot(racc[...], a_ref[g], preferred_element_type=jnp.float32)
             for g in range(NG)], axis=1)
        vblk[...] += res * f2c_ref[0, 0]

        @pl.when(j == NE - 1)
        def _():
            deg = 1.0 + vblk[...]
            dinvb = jnp.where(deg > 0.0, lax.rsqrt(deg), 0.0)
            dinv[io] = dinvb
            yt = jnp.dot(wt_ref[0], xt_ref[0],
                         preferred_element_type=jnp.float32)
            zt[io] = jnp.concatenate(
                [dinvb[0:1, :] * yt, dinvb[1:2, :] * yt], axis=0)

    def contract(scr):
        outs = []
        for g in range(NG):
            acc = None
            for kb in range(NIO):
                part = jnp.dot(scr[kb],
                               a_ref[g, kb * RB:(kb + 1) * RB, :],
                               preferred_element_type=jnp.float32)
                acc = part if acc is None else acc + part
            outs.append(acc)
        return jnp.concatenate(outs, axis=1)

    @pl.when(p == 1)
    def _():
        @pl.when(j == 0)
        def _():
            sacc[...] = jnp.zeros_like(sacc)

        sacc[...] += contract(zt) * s1_ref[0, 0]

        @pl.when(j == NE - 1)
        def _():
            t1t[io] = sacc[...]

    @pl.when(p == 2)
    def _():
        @pl.when(j == 0)
        def _():
            sacc[...] = jnp.zeros_like(sacc)

        sacc[...] += contract(t1t) * s2_ref[0, 0]

        @pl.when(j == NE - 1)
        def _():
            dinvb = dinv[io]
            dcols = jnp.concatenate(
                [jnp.broadcast_to(dinvb[0:1, :], (DH, RB)),
                 jnp.broadcast_to(dinvb[1:2, :], (DH, RB))], axis=0)
            res = jnp.maximum(
                dcols * (sacc[...] + zt[io]) + b_ref[0], 0.0)
            out_ref[0] = res.T


def _main(Aall, csall, f1all, f2call, s1all, s2all, Xtall, Wtall, ball):
    grid = (2, 3, NIO, NE)
    return pl.pallas_call(
        _main_body,
        grid=grid,
        in_specs=[
            pl.BlockSpec((NG, N, DH),
                         lambda s, p, io, j: ((s * NE + j) * NIO + io, 0, 0)),
            pl.BlockSpec((1, NE, N), lambda s, p, io, j: (s, 0, 0)),
            pl.BlockSpec((1, C, NE), lambda s, p, io, j: (s, 0, 0)),
            pl.BlockSpec((1, 1, C, 1), lambda s, p, io, j: (s, j, 0, 0)),
            pl.BlockSpec((1, 1, DS, 1), lambda s, p, io, j: (s, j, 0, 0)),
            pl.BlockSpec((1, 1, DS, 1), lambda s, p, io, j: (s, j, 0, 0)),
            pl.BlockSpec((1, DS, RB), lambda s, p, io, j: (s, 0, io)),
            pl.BlockSpec((1, DH, DS), lambda s, p, io, j: (s, 0, 0)),
            pl.BlockSpec((1, DS, 1), lambda s, p, io, j: (s, 0, 0)),
        ],
        out_specs=pl.BlockSpec((1, RB, DS), lambda s, p, io, j: (s, io, 0)),
        out_shape=jax.ShapeDtypeStruct((2, N, DS), jnp.float32),
        scratch_shapes=[
            pltpu.VMEM((C, N), jnp.float32),
            pltpu.VMEM((C, RB), jnp.float32),
            pltpu.VMEM((NIO, C, RB), jnp.float32),
            pltpu.VMEM((NIO, DS, RB), jnp.float32),
            pltpu.VMEM((NIO, DS, RB), jnp.float32),
            pltpu.VMEM((DS, RB), jnp.float32),
        ],
    )(Aall, csall, f1all, f2call, s1all, s2all, Xtall, Wtall, ball)


MB = 512


def _mlp_body(bu_ref, bv_ref, t_ref, m1a_ref, m1b_ref, b1_ref, m2_ref,
              b2_ref, m3_ref, b3_ref, bp_ref, loss_ref, lacc):
    i = pl.program_id(0)

    @pl.when(i == 0)
    def _():
        lacc[...] = jnp.zeros_like(lacc)

    h = jnp.dot(bu_ref[...], m1a_ref[...], preferred_element_type=jnp.float32)
    h += jnp.dot(bv_ref[...], m1b_ref[...], preferred_element_type=jnp.float32)
    h = jnp.maximum(h + b1_ref[...], 0.0)
    h = jnp.maximum(jnp.dot(h, m2_ref[...], preferred_element_type=jnp.float32)
                    + b2_ref[...], 0.0)
    logits = jnp.dot(h, m3_ref[...], preferred_element_type=jnp.float32) \
        + b3_ref[...]
    m = jnp.max(logits, axis=-1, keepdims=True)
    e = jnp.exp(logits - m)
    bp = e / jnp.sum(e, axis=-1, keepdims=True)
    bp_ref[...] = bp

    mm = jnp.max(bp, axis=-1, keepdims=True)
    lse = mm + jnp.log(jnp.sum(jnp.exp(bp - mm), axis=-1, keepdims=True))
    t = t_ref[...]
    bpt = bp[:, 0:1] * (1.0 - t) + bp[:, 1:2] * t
    lacc[...] += jnp.sum(lse - bpt, axis=0, keepdims=True)

    @pl.when(i == pl.num_programs(0) - 1)
    def _():
        loss_ref[...] = lacc[...] * (1.0 / P)


def _mlp(Bu, Bv, targetf, M1a, M1b, b1, M2, b2, M3, b3):
    grid = (P // MB,)
    return pl.pallas_call(
        _mlp_body,
        grid=grid,
        in_specs=[
            pl.BlockSpec((MB, DS), lambda i: (i, 0)),
            pl.BlockSpec((MB, DS), lambda i: (i, 0)),
            pl.BlockSpec((MB, 1), lambda i: (i, 0)),
            pl.BlockSpec((DS, DS), lambda i: (0, 0)),
            pl.BlockSpec((DS, DS), lambda i: (0, 0)),
            pl.BlockSpec((1, DS), lambda i: (0, 0)),
            pl.BlockSpec((DS, DS // 2), lambda i: (0, 0)),
            pl.BlockSpec((1, DS // 2), lambda i: (0, 0)),
            pl.BlockSpec((DS // 2, 2), lambda i: (0, 0)),
            pl.BlockSpec((1, 2), lambda i: (0, 0)),
        ],
        out_specs=[
            pl.BlockSpec((MB, 2), lambda i: (i, 0)),
            pl.BlockSpec((1, 1), lambda i: (0, 0)),
        ],
        out_shape=[
            jax.ShapeDtypeStruct((P, 2), jnp.float32),
            jax.ShapeDtypeStruct((1, 1), jnp.float32),
        ],
        scratch_shapes=[pltpu.VMEM((1, 1), jnp.float32)],
    )(Bu, Bv, targetf, M1a, M1b, b1, M2, b2, M3, b3)


def kernel(edge_index_u, edge_value_u, X_u, edge_index_v, edge_value_v, X_v,
           index_list, Wgt1_u, Wgt2_u, Wgt1_v, Wgt2_v, Wg_u, bg_u, Wg_v, bg_v,
           M1, b1, M2, b2, M3, b3):
    rows = jnp.concatenate([edge_index_u[:, 0, :].reshape(-1),
                            edge_index_v[:, 0, :].reshape(-1)]) \
        .astype(jnp.int32)
    cols = jnp.concatenate([edge_index_u[:, 1, :].reshape(-1),
                            edge_index_v[:, 1, :].reshape(-1)]) \
        .astype(jnp.int32)
    vals = jnp.concatenate([edge_value_u.reshape(-1),
                            edge_value_v.reshape(-1)])
    Afall, csfall = _sc_scatter(rows, cols, vals)
    Aall = Afall.reshape(2 * NE * (N // DH), N, DH)
    csall = csfall.reshape(2, NE, N)

    f1u = jax.nn.softmax(Wgt1_u, axis=1)
    f2u = jax.nn.softmax(Wgt2_u, axis=1)
    f1v = jax.nn.softmax(Wgt1_v, axis=1)
    f2v = jax.nn.softmax(Wgt2_v, axis=1)
    f1all = jnp.stack([f1u, f1v])
    f2call = jnp.stack([f2u.T, f2v.T])[..., None]
    s1all = jnp.repeat(jnp.stack([f1u.T, f1v.T]), DH, axis=2)[..., None]
    s2all = jnp.repeat(jnp.stack([f2u.T, f2v.T]), DH, axis=2)[..., None]
    Xtall = jnp.stack([X_u.T, X_v.T])
    Wtall = jnp.stack([Wg_u.T, Wg_v.T])
    ball = jnp.stack([jnp.tile(bg_u, (2,))[:, None],
                      jnp.tile(bg_v, (2,))[:, None]])

    Xout = _main(Aall, csall, f1all, f2call, s1all, s2all, Xtall, Wtall,
                 ball)
    Xu_ = Xout[0]
    Xv_ = Xout[1]

    u_idx = index_list[:, 0].astype(jnp.int32)
    v_idx = index_list[:, 1].astype(jnp.int32)
    target = index_list[:, 2]
    targetf = target.astype(jnp.float32)

    table = Xout.reshape(2 * N, DS)
    catidx = jnp.concatenate([u_idx, v_idx + N])
    Bcat = _sc_gather(table, catidx)
    Bu = Bcat[:P]
    Bv = Bcat[P:]

    Bp, loss2 = _mlp(Bu, Bv, targetf[:, None], M1[:DS], M1[DS:], b1[None, :],
                     M2, b2[None, :], M3, b3[None, :])
    loss = loss2.reshape(())
    return (Xu_, Xv_, f1u, f2u, f1v, f2v, loss, Bp, targetf)


# speedup vs baseline: 1.1676x; 1.0100x over previous
import jax
import jax.numpy as jnp
from jax import lax
from jax.experimental import pallas as pl
from jax.experimental.pallas import tpu as pltpu
from jax.experimental.pallas import tpu_sc as plsc

N = 1024
NE = 3
C = 2
E = 32768
P = 4096
DH = 128
DS = C * DH
RB = 256

NSUB = 16
ECH = E // NSUB
ZCH = 65664
ACC = ZCH * NSUB
MCH = N * N // NSUB


def _sc_scatter_body(r_hbm, c_hbm, v_hbm, zeros_hbm, a_out, cs_out,
                     acc, rbuf, cbuf, vbuf, linbuf, csbuf):
    cid = lax.axis_index("c")
    sid = lax.axis_index("s")
    ebase = cid * (NE * E)
    abase = cid * (NE * N * N)
    cbase = cid * (NE * N)
    for j in range(NE):
        pltpu.sync_copy(zeros_hbm, acc.at[pl.ds(sid * ZCH, ZCH)])
        base = ebase + j * E + sid * ECH
        pltpu.sync_copy(r_hbm.at[pl.ds(base, ECH)], rbuf)
        pltpu.sync_copy(c_hbm.at[pl.ds(base, ECH)], cbuf)
        pltpu.sync_copy(v_hbm.at[pl.ds(base, ECH)], vbuf)

        def body(i, _):
            o = i * 16
            r = rbuf[pl.ds(o, 16)]
            c = cbuf[pl.ds(o, 16)]
            linbuf[pl.ds(o, 16)] = ((c >> 7) << 17) + (r << 7) + (c & 127)
            csbuf[pl.ds(o, 16)] = c + N * N
            return 0

        lax.fori_loop(0, ECH // 16, body, 0)
        plsc.subcore_barrier()
        pltpu.sync_copy(vbuf, acc.at[linbuf], add=True)
        pltpu.sync_copy(vbuf, acc.at[csbuf], add=True)
        plsc.subcore_barrier()
        pltpu.sync_copy(acc.at[pl.ds(sid * MCH, MCH)],
                        a_out.at[pl.ds(abase + j * N * N + sid * MCH, MCH)])

        @pl.when(sid == 0)
        def _():
            pltpu.sync_copy(acc.at[pl.ds(N * N, N)],
                            cs_out.at[pl.ds(cbase + j * N, N)])

        plsc.subcore_barrier()


@jax.jit
def _sc_scatter(rows, cols, vals):
    zeros = jnp.zeros((ZCH,), jnp.float32)
    mesh = plsc.VectorSubcoreMesh(core_axis_name="c", subcore_axis_name="s")
    f = pl.kernel(
        _sc_scatter_body,
        mesh=mesh,
        out_type=[
            jax.ShapeDtypeStruct((2 * NE * N * N,), jnp.float32),
            jax.ShapeDtypeStruct((2 * NE * N,), jnp.float32),
        ],
        scratch_types=[
            pltpu.VMEM_SHARED((ACC,), jnp.float32),
            pltpu.VMEM((ECH,), jnp.int32),
            pltpu.VMEM((ECH,), jnp.int32),
            pltpu.VMEM((ECH,), jnp.float32),
            pltpu.VMEM((ECH,), jnp.int32),
            pltpu.VMEM((ECH,), jnp.int32),
        ],
    )
    return f(rows, cols, vals, zeros)


GCH = (2 * P) // (2 * NSUB)


def _sc_gather_body(tab, idx, out, ibuf, rows, sem):
    cid = lax.axis_index("c")
    sid = lax.axis_index("s")
    base = (cid * NSUB + sid) * GCH
    pltpu.sync_copy(idx.at[pl.ds(base, GCH)], ibuf)
    pltpu.async_copy(tab.at[ibuf], rows, sem).wait()
    pltpu.sync_copy(rows, out.at[pl.ds(base, GCH)])


@jax.jit
def _sc_gather(table, catidx):
    mesh = plsc.VectorSubcoreMesh(core_axis_name="c", subcore_axis_name="s")
    f = pl.kernel(
        _sc_gather_body,
        mesh=mesh,
        out_type=jax.ShapeDtypeStruct((2 * P, DS), jnp.float32),
        scratch_types=[
            pltpu.VMEM((GCH,), jnp.int32),
            pltpu.VMEM((GCH, DS), jnp.float32),
            pltpu.SemaphoreType.DMA,
        ],
    )
    return f(table, catidx)


NIO = N // RB
NG = RB // DH


def _main_body(a_ref, cs_ref, f1_ref, f2c_ref, s1_ref, s2_ref, xt_ref,
               wt_ref, b_ref, out_ref, racc, vblk, dinv, zt, t1t, sacc):
    p = pl.program_id(1)
    io = pl.program_id(2)
    j = pl.program_id(3)

    @pl.when((p == 0) & (io == 0) & (j == 0))
    def _():
        racc[...] = jnp.dot(f1_ref[0], cs_ref[0],
                            preferred_element_type=jnp.float32)

    @pl.when(p == 0)
    def _():
        @pl.when(j == 0)
        def _():
            vblk[...] = jnp.zeros_like(vblk)

        res = jnp.concatenate(
            [jnp.dot(racc[...], a_ref[g], preferred_element_type=jnp.float32)
             for g in range(NG)], axis=1)
        vblk[...] += res * f2c_ref[0, 0]

        @pl.when(j == NE - 1)
        def _():
            deg = 1.0 + vblk[...]
            dinvb = jnp.where(deg > 0.0, lax.rsqrt(deg), 0.0)
            dinv[io] = dinvb
            yt = jnp.dot(wt_ref[0], xt_ref[0],
                         preferred_element_type=jnp.float32)
            zt[io] = jnp.concatenate(
                [dinvb[0:1, :] * yt, dinvb[1:2, :] * yt], axis=0)

    def contract(scr):
        outs = []
        for g in range(NG):
            acc = None
            for kb in range(NIO):
                part = jnp.dot(scr[kb],
                               a_ref[g, kb * RB:(kb + 1) * RB, :],
                               preferred_element_type=jnp.float32)
                acc = part if acc is None else acc + part
            outs.append(acc)
        return jnp.concatenate(outs, axis=1)

    @pl.when(p == 1)
    def _():
        @pl.when(j == 0)
        def _():
            sacc[...] = jnp.zeros_like(sacc)

        sacc[...] += contract(zt) * s1_ref[0, 0]

        @pl.when(j == NE - 1)
        def _():
            t1t[io] = sacc[...]

    @pl.when(p == 2)
    def _():
        @pl.when(j == 0)
        def _():
            sacc[...] = jnp.zeros_like(sacc)

        sacc[...] += contract(t1t) * s2_ref[0, 0]

        @pl.when(j == NE - 1)
        def _():
            dinvb = dinv[io]
            dcols = jnp.concatenate(
                [jnp.broadcast_to(dinvb[0:1, :], (DH, RB)),
                 jnp.broadcast_to(dinvb[1:2, :], (DH, RB))], axis=0)
            res = jnp.maximum(
                dcols * (sacc[...] + zt[io]) + b_ref[0], 0.0)
            out_ref[0] = res.T


def _main(Aall, csall, f1all, f2call, s1all, s2all, Xtall, Wtall, ball):
    grid = (2, 3, NIO, NE)
    return pl.pallas_call(
        _main_body,
        grid=grid,
        in_specs=[
            pl.BlockSpec((NG, N, DH),
                         lambda s, p, io, j: ((s * NE + j) * NIO + io, 0, 0)),
            pl.BlockSpec((1, NE, N), lambda s, p, io, j: (s, 0, 0)),
            pl.BlockSpec((1, C, NE), lambda s, p, io, j: (s, 0, 0)),
            pl.BlockSpec((1, 1, C, 1), lambda s, p, io, j: (s, j, 0, 0)),
            pl.BlockSpec((1, 1, DS, 1), lambda s, p, io, j: (s, j, 0, 0)),
            pl.BlockSpec((1, 1, DS, 1), lambda s, p, io, j: (s, j, 0, 0)),
            pl.BlockSpec((1, DS, RB), lambda s, p, io, j: (s, 0, io)),
            pl.BlockSpec((1, DH, DS), lambda s, p, io, j: (s, 0, 0)),
            pl.BlockSpec((1, DS, 1), lambda s, p, io, j: (s, 0, 0)),
        ],
        out_specs=pl.BlockSpec((1, RB, DS), lambda s, p, io, j: (s, io, 0)),
        out_shape=jax.ShapeDtypeStruct((2, N, DS), jnp.float32),
        scratch_shapes=[
            pltpu.VMEM((C, N), jnp.float32),
            pltpu.VMEM((C, RB), jnp.float32),
            pltpu.VMEM((NIO, C, RB), jnp.float32),
            pltpu.VMEM((NIO, DS, RB), jnp.float32),
            pltpu.VMEM((NIO, DS, RB), jnp.float32),
            pltpu.VMEM((DS, RB), jnp.float32),
        ],
    )(Aall, csall, f1all, f2call, s1all, s2all, Xtall, Wtall, ball)


MB = 1024


def _mlp_body(bu_ref, bv_ref, t_ref, m1a_ref, m1b_ref, b1_ref, m2_ref,
              b2_ref, m3_ref, b3_ref, bp_ref, loss_ref, lacc):
    i = pl.program_id(0)

    @pl.when(i == 0)
    def _():
        lacc[...] = jnp.zeros_like(lacc)

    h = jnp.dot(bu_ref[...], m1a_ref[...], preferred_element_type=jnp.float32)
    h += jnp.dot(bv_ref[...], m1b_ref[...], preferred_element_type=jnp.float32)
    h = jnp.maximum(h + b1_ref[...], 0.0)
    h = jnp.maximum(jnp.dot(h, m2_ref[...], preferred_element_type=jnp.float32)
                    + b2_ref[...], 0.0)
    logits = jnp.dot(h, m3_ref[...], preferred_element_type=jnp.float32) \
        + b3_ref[...]
    m = jnp.max(logits, axis=-1, keepdims=True)
    e = jnp.exp(logits - m)
    bp = e / jnp.sum(e, axis=-1, keepdims=True)
    bp_ref[...] = bp

    mm = jnp.max(bp, axis=-1, keepdims=True)
    lse = mm + jnp.log(jnp.sum(jnp.exp(bp - mm), axis=-1, keepdims=True))
    t = t_ref[...]
    bpt = bp[:, 0:1] * (1.0 - t) + bp[:, 1:2] * t
    lacc[...] += jnp.sum(lse - bpt, axis=0, keepdims=True)

    @pl.when(i == pl.num_programs(0) - 1)
    def _():
        loss_ref[...] = lacc[...] * (1.0 / P)


def _mlp(Bu, Bv, targetf, M1a, M1b, b1, M2, b2, M3, b3):
    grid = (P // MB,)
    return pl.pallas_call(
        _mlp_body,
        grid=grid,
        in_specs=[
            pl.BlockSpec((MB, DS), lambda i: (i, 0)),
            pl.BlockSpec((MB, DS), lambda i: (i, 0)),
            pl.BlockSpec((MB, 1), lambda i: (i, 0)),
            pl.BlockSpec((DS, DS), lambda i: (0, 0)),
            pl.BlockSpec((DS, DS), lambda i: (0, 0)),
            pl.BlockSpec((1, DS), lambda i: (0, 0)),
            pl.BlockSpec((DS, DS // 2), lambda i: (0, 0)),
            pl.BlockSpec((1, DS // 2), lambda i: (0, 0)),
            pl.BlockSpec((DS // 2, 2), lambda i: (0, 0)),
            pl.BlockSpec((1, 2), lambda i: (0, 0)),
        ],
        out_specs=[
            pl.BlockSpec((MB, 2), lambda i: (i, 0)),
            pl.BlockSpec((1, 1), lambda i: (0, 0)),
        ],
        out_shape=[
            jax.ShapeDtypeStruct((P, 2), jnp.float32),
            jax.ShapeDtypeStruct((1, 1), jnp.float32),
        ],
        scratch_shapes=[pltpu.VMEM((1, 1), jnp.float32)],
    )(Bu, Bv, targetf, M1a, M1b, b1, M2, b2, M3, b3)


def kernel(edge_index_u, edge_value_u, X_u, edge_index_v, edge_value_v, X_v,
           index_list, Wgt1_u, Wgt2_u, Wgt1_v, Wgt2_v, Wg_u, bg_u, Wg_v, bg_v,
           M1, b1, M2, b2, M3, b3):
    rows = jnp.concatenate([edge_index_u[:, 0, :].reshape(-1),
                            edge_index_v[:, 0, :].reshape(-1)]) \
        .astype(jnp.int32)
    cols = jnp.concatenate([edge_index_u[:, 1, :].reshape(-1),
                            edge_index_v[:, 1, :].reshape(-1)]) \
        .astype(jnp.int32)
    vals = jnp.concatenate([edge_value_u.reshape(-1),
                            edge_value_v.reshape(-1)])
    Afall, csfall = _sc_scatter(rows, cols, vals)
    Aall = Afall.reshape(2 * NE * (N // DH), N, DH)
    csall = csfall.reshape(2, NE, N)

    f1u = jax.nn.softmax(Wgt1_u, axis=1)
    f2u = jax.nn.softmax(Wgt2_u, axis=1)
    f1v = jax.nn.softmax(Wgt1_v, axis=1)
    f2v = jax.nn.softmax(Wgt2_v, axis=1)
    f1all = jnp.stack([f1u, f1v])
    f2call = jnp.stack([f2u.T, f2v.T])[..., None]
    s1all = jnp.repeat(jnp.stack([f1u.T, f1v.T]), DH, axis=2)[..., None]
    s2all = jnp.repeat(jnp.stack([f2u.T, f2v.T]), DH, axis=2)[..., None]
    Xtall = jnp.stack([X_u.T, X_v.T])
    Wtall = jnp.stack([Wg_u.T, Wg_v.T])
    ball = jnp.stack([jnp.tile(bg_u, (2,))[:, None],
                      jnp.tile(bg_v, (2,))[:, None]])

    Xout = _main(Aall, csall, f1all, f2call, s1all, s2all, Xtall, Wtall,
                 ball)
    Xu_ = Xout[0]
    Xv_ = Xout[1]

    u_idx = index_list[:, 0].astype(jnp.int32)
    v_idx = index_list[:, 1].astype(jnp.int32)
    target = index_list[:, 2]
    targetf = target.astype(jnp.float32)

    table = Xout.reshape(2 * N, DS)
    catidx = jnp.concatenate([u_idx, v_idx + N])
    Bcat = _sc_gather(table, catidx)
    Bu = Bcat[:P]
    Bv = Bcat[P:]

    Bp, loss2 = _mlp(Bu, Bv, targetf[:, None], M1[:DS], M1[DS:], b1[None, :],
                     M2, b2[None, :], M3, b3[None, :])
    loss = loss2.reshape(())
    return (Xu_, Xv_, f1u, f2u, f1v, f2v, loss, Bp, targetf)
